# baseline (speedup 1.0000x reference)
import functools

import jax
import jax.numpy as jnp
from jax.experimental import pallas as pl
from jax.experimental.pallas import tpu as pltpu

_N_CODEBOOKS = 8
_CODEBOOK_SIZE = 1024
_LATENT_DIM = 64
_BLOCK = 1024


def _split3(w):
    b = jax.lax.bitcast_convert_type(w, jnp.uint32)
    w1 = jax.lax.bitcast_convert_type(b & jnp.uint32(0xFFFF0000), jnp.float32)
    r1 = w - w1
    b1 = jax.lax.bitcast_convert_type(r1, jnp.uint32)
    w2 = jax.lax.bitcast_convert_type(b1 & jnp.uint32(0xFFFF0000), jnp.float32)
    w3 = r1 - w2
    return (
        w1.astype(jnp.bfloat16),
        w2.astype(jnp.bfloat16),
        w3.astype(jnp.bfloat16),
    )


def _half_stage(r, w, wcat, x2, y2):
    n2p = jax.lax.dot_general(
        jnp.float32(-2.0) * r, w, (((1,), (1,)), ((), ())),
        preferred_element_type=jnp.float32,
    )
    d2 = (x2 + y2) + n2p
    d = jnp.sqrt(jnp.maximum(d2, 0.0))
    m = jnp.min(d, axis=1, keepdims=True)
    iota = jax.lax.broadcasted_iota(jnp.int32, d2.shape, 1)
    idxv = jnp.where(d == m, iota, jnp.int32(_CODEBOOK_SIZE))
    idx = jnp.min(idxv, axis=1, keepdims=True)
    mask = iota == idx
    onehot = mask.astype(jnp.float32).astype(jnp.bfloat16)
    qcat = jax.lax.dot_general(
        onehot, wcat, (((1,), (0,)), ((), ())),
        preferred_element_type=jnp.float32,
    )
    dd = _LATENT_DIM
    q = (qcat[:, :dd] + qcat[:, dd:2 * dd]) + qcat[:, 2 * dd:]
    return idx, q


def _stage_kernel(r_ref, w_ref, x2_ref, y2_ref, idx_ref, q_ref):
    w = w_ref[...]
    y2 = y2_ref[...]
    w1, w2, w3 = _split3(w)
    wcat = jnp.concatenate([w1, w2, w3], axis=1)
    h = _BLOCK // 4
    for s in range(4):
        sl = pl.ds(s * h, h)
        idx, q = _half_stage(r_ref[sl, :], w, wcat, x2_ref[sl, :], y2)
        idx_ref[sl, :] = idx
        q_ref[sl, :] = q


def _stage(rf, w, x2, y2):
    n_tokens, dim = rf.shape
    n_blocks = n_tokens // _BLOCK
    idx, q = pl.pallas_call(
        _stage_kernel,
        grid=(n_blocks,),
        in_specs=[
            pl.BlockSpec((_BLOCK, dim), lambda i: (i, 0)),
            pl.BlockSpec((_CODEBOOK_SIZE, _LATENT_DIM), lambda i: (0, 0)),
            pl.BlockSpec((_BLOCK, 1), lambda i: (i, 0)),
            pl.BlockSpec((1, _CODEBOOK_SIZE), lambda i: (0, 0)),
        ],
        out_specs=[
            pl.BlockSpec((_BLOCK, 1), lambda i: (i, 0)),
            pl.BlockSpec((_BLOCK, dim), lambda i: (i, 0)),
        ],
        out_shape=[
            jax.ShapeDtypeStruct((n_tokens, 1), jnp.int32),
            jax.ShapeDtypeStruct((n_tokens, dim), jnp.float32),
        ],
        compiler_params=pltpu.CompilerParams(
            dimension_semantics=("parallel",),
        ),
    )(rf, w, x2, y2)
    return idx[:, 0], q


@jax.jit
def kernel(z, codebooks):
    residual = z
    quantized = jnp.zeros_like(z)
    indices_list = []
    commitment_loss = jnp.asarray(0.0, dtype=z.dtype)
    for i in range(_N_CODEBOOKS):
        weight = codebooks[i]
        residual_flat = residual.reshape(-1, residual.shape[-1])
        x2 = jnp.sum(residual_flat * residual_flat, axis=-1, keepdims=True)
        y2 = jnp.sum(weight * weight, axis=-1)[None, :]
        idx, qf = _stage(residual_flat, weight, x2, y2)
        indices_list.append(idx)
        quantized_i = qf.reshape(residual.shape)
        quantized = quantized + quantized_i
        residual = residual - quantized_i
        commitment_loss = commitment_loss + jnp.mean(residual ** 2)
    indices_stack = jnp.stack(indices_list, axis=-1)
    return quantized, indices_stack, commitment_loss


# speedup vs baseline: 1.0062x; 1.0062x over previous
import functools

import jax
import jax.numpy as jnp
from jax.experimental import pallas as pl
from jax.experimental.pallas import tpu as pltpu

_N_CODEBOOKS = 8
_CODEBOOK_SIZE = 1024
_LATENT_DIM = 64
_BLOCK = 1024


def _split3(w):
    b = jax.lax.bitcast_convert_type(w, jnp.uint32)
    w1 = jax.lax.bitcast_convert_type(b & jnp.uint32(0xFFFF0000), jnp.float32)
    r1 = w - w1
    b1 = jax.lax.bitcast_convert_type(r1, jnp.uint32)
    w2 = jax.lax.bitcast_convert_type(b1 & jnp.uint32(0xFFFF0000), jnp.float32)
    w3 = r1 - w2
    return (
        w1.astype(jnp.bfloat16),
        w2.astype(jnp.bfloat16),
        w3.astype(jnp.bfloat16),
    )


def _half_stage(r, w, wcat, x2, y2):
    n2p = jax.lax.dot_general(
        jnp.float32(-2.0) * r, w, (((1,), (1,)), ((), ())),
        preferred_element_type=jnp.float32,
    )
    d2 = (x2 + y2) + n2p
    d = jnp.sqrt(jnp.maximum(d2, 0.0))
    m = jnp.min(d, axis=1, keepdims=True)
    iota = jax.lax.broadcasted_iota(jnp.int32, d2.shape, 1)
    idxv = jnp.where(d == m, iota, jnp.int32(_CODEBOOK_SIZE))
    idx = jnp.min(idxv, axis=1, keepdims=True)
    mask = iota == idx
    onehot = mask.astype(jnp.bfloat16)
    qcat = jax.lax.dot_general(
        onehot, wcat, (((1,), (0,)), ((), ())),
        preferred_element_type=jnp.float32,
    )
    dd = _LATENT_DIM
    q = (qcat[:, :dd] + qcat[:, dd:2 * dd]) + qcat[:, 2 * dd:]
    return idx, q


def _stage_kernel(r_ref, w_ref, x2_ref, y2_ref, idx_ref, q_ref):
    w = w_ref[...]
    y2 = y2_ref[...]
    w1, w2, w3 = _split3(w)
    wcat = jnp.concatenate([w1, w2, w3], axis=1)
    h = _BLOCK // 2
    for s in range(2):
        sl = pl.ds(s * h, h)
        idx, q = _half_stage(r_ref[sl, :], w, wcat, x2_ref[sl, :], y2)
        idx_ref[sl, :] = idx
        q_ref[sl, :] = q


def _stage(rf, w, x2, y2):
    n_tokens, dim = rf.shape
    n_blocks = n_tokens // _BLOCK
    idx, q = pl.pallas_call(
        _stage_kernel,
        grid=(n_blocks,),
        in_specs=[
            pl.BlockSpec((_BLOCK, dim), lambda i: (i, 0)),
            pl.BlockSpec((_CODEBOOK_SIZE, _LATENT_DIM), lambda i: (0, 0)),
            pl.BlockSpec((_BLOCK, 1), lambda i: (i, 0)),
            pl.BlockSpec((1, _CODEBOOK_SIZE), lambda i: (0, 0)),
        ],
        out_specs=[
            pl.BlockSpec((_BLOCK, 1), lambda i: (i, 0)),
            pl.BlockSpec((_BLOCK, dim), lambda i: (i, 0)),
        ],
        out_shape=[
            jax.ShapeDtypeStruct((n_tokens, 1), jnp.int32),
            jax.ShapeDtypeStruct((n_tokens, dim), jnp.float32),
        ],
        compiler_params=pltpu.CompilerParams(
            dimension_semantics=("parallel",),
        ),
    )(rf, w, x2, y2)
    return idx[:, 0], q


@jax.jit
def kernel(z, codebooks):
    residual = z
    quantized = jnp.zeros_like(z)
    indices_list = []
    commitment_loss = jnp.asarray(0.0, dtype=z.dtype)
    for i in range(_N_CODEBOOKS):
        weight = codebooks[i]
        residual_flat = residual.reshape(-1, residual.shape[-1])
        x2 = jnp.sum(residual_flat * residual_flat, axis=-1, keepdims=True)
        y2 = jnp.sum(weight * weight, axis=-1)[None, :]
        idx, qf = _stage(residual_flat, weight, x2, y2)
        indices_list.append(idx)
        quantized_i = qf.reshape(residual.shape)
        quantized = quantized + quantized_i
        residual = residual - quantized_i
        commitment_loss = commitment_loss + jnp.mean(residual ** 2)
    indices_stack = jnp.stack(indices_list, axis=-1)
    return quantized, indices_stack, commitment_loss


# speedup vs baseline: 1.0069x; 1.0007x over previous
import jax
import jax.numpy as jnp
from jax.experimental import pallas as pl
from jax.experimental.pallas import tpu as pltpu

_N_CODEBOOKS = 8
_CODEBOOK_SIZE = 1024
_LATENT_DIM = 64
_BLOCK = 1024


def _split3(w):
    b = jax.lax.bitcast_convert_type(w, jnp.uint32)
    w1 = jax.lax.bitcast_convert_type(b & jnp.uint32(0xFFFF0000), jnp.float32)
    r1 = w - w1
    b1 = jax.lax.bitcast_convert_type(r1, jnp.uint32)
    w2 = jax.lax.bitcast_convert_type(b1 & jnp.uint32(0xFFFF0000), jnp.float32)
    w3 = r1 - w2
    return (
        w1.astype(jnp.bfloat16),
        w2.astype(jnp.bfloat16),
        w3.astype(jnp.bfloat16),
    )


def _half_stage(r, w, wcat, x2, y2):
    n2p = jax.lax.dot_general(
        jnp.float32(-2.0) * r, w, (((1,), (1,)), ((), ())),
        preferred_element_type=jnp.float32,
    )
    d2 = (x2 + y2) + n2p
    d = jnp.sqrt(jnp.maximum(d2, 0.0))
    m = jnp.min(d, axis=1, keepdims=True)
    iota = jax.lax.broadcasted_iota(jnp.int32, d2.shape, 1)
    idxv = jnp.where(d == m, iota, jnp.int32(_CODEBOOK_SIZE))
    idx = jnp.min(idxv, axis=1, keepdims=True)
    mask = iota == idx
    onehot = mask.astype(jnp.bfloat16)
    qcat = jax.lax.dot_general(
        onehot, wcat, (((1,), (0,)), ((), ())),
        preferred_element_type=jnp.float32,
    )
    dd = _LATENT_DIM
    q = (qcat[:, :dd] + qcat[:, dd:2 * dd]) + qcat[:, 2 * dd:]
    return idx, q


def _stage_kernel(r_ref, w_ref, x2_ref, y2_ref, idx_ref, q_ref):
    w = w_ref[...]
    y2 = y2_ref[...]
    w1, w2, w3 = _split3(w)
    wcat = jnp.concatenate([w1, w2, w3], axis=1)
    h = _BLOCK // 2
    for s in range(2):
        sl = pl.ds(s * h, h)
        idx, q = _half_stage(r_ref[sl, :], w, wcat, x2_ref[sl, :], y2)
        idx_ref[sl, :] = idx
        q_ref[sl, :] = q


def _stage(rf, w, x2, y2):
    n_tokens, dim = rf.shape
    n_blocks = n_tokens // _BLOCK
    idx, q = pl.pallas_call(
        _stage_kernel,
        grid=(n_blocks,),
        in_specs=[
            pl.BlockSpec((_BLOCK, dim), lambda i: (i, 0)),
            pl.BlockSpec((_CODEBOOK_SIZE, _LATENT_DIM), lambda i: (0, 0)),
            pl.BlockSpec((_BLOCK, 1), lambda i: (i, 0)),
            pl.BlockSpec((1, _CODEBOOK_SIZE), lambda i: (0, 0)),
        ],
        out_specs=[
            pl.BlockSpec((_BLOCK, 1), lambda i: (i, 0)),
            pl.BlockSpec((_BLOCK, dim), lambda i: (i, 0)),
        ],
        out_shape=[
            jax.ShapeDtypeStruct((n_tokens, 1), jnp.int32),
            jax.ShapeDtypeStruct((n_tokens, dim), jnp.float32),
        ],
        compiler_params=pltpu.CompilerParams(
            dimension_semantics=("parallel",),
        ),
    )(rf, w, x2, y2)
    return idx[:, 0], q


@jax.jit
def kernel(z, codebooks):
    residual = z
    quantized = jnp.zeros_like(z)
    indices_list = []
    commitment_loss = jnp.asarray(0.0, dtype=z.dtype)
    for i in range(_N_CODEBOOKS):
        weight = codebooks[i]
        residual_flat = residual.reshape(-1, residual.shape[-1])
        x2 = jnp.sum(residual_flat * residual_flat, axis=-1, keepdims=True)
        y2 = jnp.sum(weight * weight, axis=-1)[None, :]
        idx, qf = _stage(residual_flat, weight, x2, y2)
        indices_list.append(idx)
        quantized_i = qf.reshape(residual.shape)
        quantized = quantized + quantized_i
        residual = residual - quantized_i
        commitment_loss = commitment_loss + jnp.mean(residual ** 2)
    indices_stack = jnp.stack(indices_list, axis=-1)
    return quantized, indices_stack, commitment_loss
